# Initial kernel scaffold; baseline (speedup 1.0000x reference)
import functools

import jax
import jax.numpy as jnp
from jax import lax
from jax.experimental import pallas as pl
from jax.experimental.pallas import tpu as pltpu
from jax.experimental.pallas import tpu_sc as plsc

NU = 5000
NI = 5000
R = 5
IN_FEAT = 128
HID = 128
HALF = 64

NC = 2
NS = 16
CHUNK = 128

ACC_ROWS = R * NU + 8
ROWS_PT = ACC_ROWS // NS


def _sc_accumulate(xi1, xi2, xu1, xu2, gu, gv, ku, kv, ones_i, zf, zc):
    nchunk = gu.shape[1]

    def body(xi1_h, xi2_h, xu1_h, xu2_h, gu_h, gv_h, ku_h, kv_h, ones_h, zf_h, zc_h,
             ou1, ou2, ouc, ov1, ov2, ovc,
             keys_t, gidx_t, rows_t, ones_t, acc_f, acc_c, sem):
        cid = lax.axis_index("c")
        sid = lax.axis_index("s")
        base = sid * ROWS_PT

        def run_side(tab1, tab2, gidx_hbm, keys_hbm, o1, o2, oc):
            pltpu.sync_copy(keys_hbm.at[sid], keys_t)
            pltpu.sync_copy(gidx_hbm.at[sid], gidx_t)
            pltpu.sync_copy(ones_h, ones_t)
            pltpu.sync_copy(zf_h, acc_f.at[pl.ds(base, ROWS_PT)])
            pltpu.sync_copy(zc_h, acc_c.at[pl.ds(base, ROWS_PT)])
            plsc.subcore_barrier()

            def chunk1(j, carry):
                pltpu.async_copy(tab1.at[gidx_t.at[j]], rows_t, sem).wait()
                pltpu.sync_copy(ones_t, acc_c.at[keys_t.at[j]], add=True)
                pltpu.sync_copy(rows_t, acc_f.at[keys_t.at[j]], add=True)
                return carry

            lax.fori_loop(0, nchunk, chunk1, 0)
            plsc.subcore_barrier()
            pltpu.sync_copy(acc_f.at[pl.ds(base, ROWS_PT)], o1.at[pl.ds(base, ROWS_PT)])
            pltpu.sync_copy(acc_c.at[pl.ds(base, ROWS_PT)], oc.at[pl.ds(base, ROWS_PT)])
            pltpu.sync_copy(zf_h, acc_f.at[pl.ds(base, ROWS_PT)])
            plsc.subcore_barrier()

            def chunk2(j, carry):
                pltpu.async_copy(tab2.at[gidx_t.at[j]], rows_t, sem).wait()
                pltpu.sync_copy(rows_t, acc_f.at[keys_t.at[j]], add=True)
                return carry

            lax.fori_loop(0, nchunk, chunk2, 0)
            plsc.subcore_barrier()
            pltpu.sync_copy(acc_f.at[pl.ds(base, ROWS_PT)], o2.at[pl.ds(base, ROWS_PT)])

        @pl.when(cid == 0)
        def _():
            run_side(xi1_h, xi2_h, gu_h, ku_h, ou1, ou2, ouc)

        @pl.when(cid == 1)
        def _():
            run_side(xu1_h, xu2_h, gv_h, kv_h, ov1, ov2, ovc)

    mesh = plsc.VectorSubcoreMesh(
        core_axis_name="c", subcore_axis_name="s", num_cores=NC, num_subcores=NS)
    f = pl.kernel(
        body,
        out_type=[
            jax.ShapeDtypeStruct((ACC_ROWS, HALF), jnp.float32),
            jax.ShapeDtypeStruct((ACC_ROWS, HALF), jnp.float32),
            jax.ShapeDtypeStruct((ACC_ROWS, 16), jnp.float32),
            jax.ShapeDtypeStruct((ACC_ROWS, HALF), jnp.float32),
            jax.ShapeDtypeStruct((ACC_ROWS, HALF), jnp.float32),
            jax.ShapeDtypeStruct((ACC_ROWS, 16), jnp.float32),
        ],
        mesh=mesh,
        scratch_types=[
            pltpu.VMEM((nchunk, CHUNK), jnp.int32),
            pltpu.VMEM((nchunk, CHUNK), jnp.int32),
            pltpu.VMEM((CHUNK, HALF), jnp.float32),
            pltpu.VMEM((CHUNK, 16), jnp.float32),
            pltpu.VMEM_SHARED((ACC_ROWS, HALF), jnp.float32),
            pltpu.VMEM_SHARED((ACC_ROWS, 16), jnp.float32),
            pltpu.SemaphoreType.DMA,
        ],
    )
    return f(xi1, xi2, xu1, xu2, gu, gv, ku, kv, ones_i, zf, zc)


def _tc_body(f1_ref, f2_ref, cnt_ref, w_ref, out_ref):
    acc = jnp.zeros(out_ref.shape, jnp.float32)
    for r in range(R):
        c = cnt_ref[r, :, 0:1]
        rc = jnp.where(c > 0.5, 1.0 / c, 0.0)
        acc = acc + jnp.dot(f1_ref[r] * rc, w_ref[r, :HALF, :],
                            preferred_element_type=jnp.float32)
        acc = acc + jnp.dot(f2_ref[r] * rc, w_ref[r, HALF:, :],
                            preferred_element_type=jnp.float32)
    out_ref[...] = acc


def _tc_project(f1, f2, cnt, weight, n):
    blk = 1000
    grid = (n // blk,)
    return pl.pallas_call(
        _tc_body,
        grid=grid,
        in_specs=[
            pl.BlockSpec((R, blk, HALF), lambda g: (0, g, 0)),
            pl.BlockSpec((R, blk, HALF), lambda g: (0, g, 0)),
            pl.BlockSpec((R, blk, 16), lambda g: (0, g, 0)),
            pl.BlockSpec((R, IN_FEAT, HID), lambda g: (0, 0, 0)),
        ],
        out_specs=pl.BlockSpec((blk, HID), lambda g: (g, 0)),
        out_shape=jax.ShapeDtypeStruct((n, HID), jnp.float32),
    )(f1, f2, cnt, weight)


def kernel(x_user, x_item, u_s, v_s, rate, weight):
    E = u_s.shape[0]
    nchunk = -(-E // (NS * CHUNK))
    e_pad = NS * nchunk * CHUNK
    pad = e_pad - E

    u_s = u_s.astype(jnp.int32)
    v_s = v_s.astype(jnp.int32)
    rate = rate.astype(jnp.int32)

    key_u = rate * NU + u_s
    key_v = rate * NI + v_s
    trash = jnp.full((pad,), R * NU, jnp.int32)
    zero_idx = jnp.zeros((pad,), jnp.int32)
    ku = jnp.concatenate([key_u, trash]).reshape(NS, nchunk, CHUNK)
    kv = jnp.concatenate([key_v, trash]).reshape(NS, nchunk, CHUNK)
    gu = jnp.concatenate([v_s, zero_idx]).reshape(NS, nchunk, CHUNK)
    gv = jnp.concatenate([u_s, zero_idx]).reshape(NS, nchunk, CHUNK)

    xi1 = x_item[:, :HALF]
    xi2 = x_item[:, HALF:]
    xu1 = x_user[:, :HALF]
    xu2 = x_user[:, HALF:]
    ones_i = jnp.ones((CHUNK, 16), jnp.float32)
    zf = jnp.zeros((ROWS_PT, HALF), jnp.float32)
    zc = jnp.zeros((ROWS_PT, 16), jnp.float32)

    ou1, ou2, ouc, ov1, ov2, ovc = _sc_accumulate(
        xi1, xi2, xu1, xu2, gu, gv, ku, kv, ones_i, zf, zc)

    fu1 = ou1[:R * NU].reshape(R, NU, HALF)
    fu2 = ou2[:R * NU].reshape(R, NU, HALF)
    cu = ouc[:R * NU].reshape(R, NU, 16)
    fv1 = ov1[:R * NI].reshape(R, NI, HALF)
    fv2 = ov2[:R * NI].reshape(R, NI, HALF)
    cv = ovc[:R * NI].reshape(R, NI, 16)

    h_u = _tc_project(fu1, fu2, cu, weight, NU)
    h_v = _tc_project(fv1, fv2, cv, weight, NI)
    return h_u, h_v


# speedup vs baseline: 15.0516x; 15.0516x over previous
import functools

import jax
import jax.numpy as jnp
from jax import lax
from jax.experimental import pallas as pl
from jax.experimental.pallas import tpu as pltpu
from jax.experimental.pallas import tpu_sc as plsc

NU = 5000
NI = 5000
R = 5
IN_FEAT = 128
HID = 128
HALF = 64
QCOL = 32
NQ = IN_FEAT // QCOL

NC = 2
NS = 16
CHUNK = 128

ACC_ROWS = R * NU + 88
ROWS_PT = ACC_ROWS // NS


def _sc_mesh():
    return plsc.VectorSubcoreMesh(
        core_axis_name="c", subcore_axis_name="s", num_cores=NC, num_subcores=NS)


def _sc_counts(ku, kv, ones_i, zc):
    nchunk = ku.shape[1]

    def body(ku_h, kv_h, ones_h, zc_h, ouc, ovc, keys_t, ones_t, acc_c):
        cid = lax.axis_index("c")
        sid = lax.axis_index("s")
        base = sid * ROWS_PT

        def run_side(keys_hbm, oc):
            pltpu.sync_copy(keys_hbm.at[sid], keys_t)
            pltpu.sync_copy(ones_h, ones_t)
            pltpu.sync_copy(zc_h, acc_c.at[pl.ds(base, ROWS_PT)])
            plsc.subcore_barrier()

            def chunk0(j, carry):
                pltpu.sync_copy(ones_t, acc_c.at[keys_t.at[j]], add=True)
                return carry

            lax.fori_loop(0, nchunk, chunk0, 0)
            plsc.subcore_barrier()
            pltpu.sync_copy(acc_c.at[pl.ds(base, ROWS_PT)], oc.at[pl.ds(base, ROWS_PT)])

        @pl.when(cid == 0)
        def _():
            run_side(ku_h, ouc)

        @pl.when(cid == 1)
        def _():
            run_side(kv_h, ovc)

    f = pl.kernel(
        body,
        out_type=[
            jax.ShapeDtypeStruct((ACC_ROWS, 16), jnp.float32),
            jax.ShapeDtypeStruct((ACC_ROWS, 16), jnp.float32),
        ],
        mesh=_sc_mesh(),
        scratch_types=[
            pltpu.VMEM((nchunk, CHUNK), jnp.int32),
            pltpu.VMEM((CHUNK, 16), jnp.float32),
            pltpu.VMEM_SHARED((ACC_ROWS, 16), jnp.float32),
        ],
        compiler_params=pltpu.CompilerParams(use_tc_tiling_on_sc=False),
    )
    return f(ku, kv, ones_i, zc)


def _sc_feats(xi_qs, xu_qs, gu, gv, ku, kv, zf):
    nchunk = gu.shape[1]
    nq = len(xi_qs)

    def body(*refs):
        xi_t = refs[0:nq]
        xu_t = refs[nq:2 * nq]
        gu_h, gv_h, ku_h, kv_h, zf_h = refs[2 * nq:2 * nq + 5]
        ou, ov = refs[2 * nq + 5:2 * nq + 7]
        keys_t, gidx_t, rows_t, acc_f, sem = refs[2 * nq + 7:]
        cid = lax.axis_index("c")
        sid = lax.axis_index("s")
        base = sid * ROWS_PT

        def run_side(tabs, gidx_hbm, keys_hbm, o):
            pltpu.sync_copy(keys_hbm.at[sid], keys_t)
            pltpu.sync_copy(gidx_hbm.at[sid], gidx_t)

            for q in range(nq):
                tab = tabs[q]
                pltpu.sync_copy(zf_h, acc_f.at[pl.ds(base, ROWS_PT)])
                plsc.subcore_barrier()

                def chunk(j, carry):
                    pltpu.async_copy(tab.at[gidx_t.at[j]], rows_t, sem).wait()
                    pltpu.sync_copy(rows_t, acc_f.at[keys_t.at[j]], add=True)
                    return carry

                lax.fori_loop(0, nchunk, chunk, 0)
                plsc.subcore_barrier()
                pltpu.sync_copy(acc_f.at[pl.ds(base, ROWS_PT)],
                                o.at[q, pl.ds(base, ROWS_PT)])
                plsc.subcore_barrier()

        @pl.when(cid == 0)
        def _():
            run_side(xi_t, gu_h, ku_h, ou)

        @pl.when(cid == 1)
        def _():
            run_side(xu_t, gv_h, kv_h, ov)

    f = pl.kernel(
        body,
        out_type=[
            jax.ShapeDtypeStruct((nq, ACC_ROWS, QCOL), jnp.float32),
            jax.ShapeDtypeStruct((nq, ACC_ROWS, QCOL), jnp.float32),
        ],
        mesh=_sc_mesh(),
        scratch_types=[
            pltpu.VMEM((nchunk, CHUNK), jnp.int32),
            pltpu.VMEM((nchunk, CHUNK), jnp.int32),
            pltpu.VMEM((CHUNK, QCOL), jnp.float32),
            pltpu.VMEM_SHARED((ACC_ROWS, QCOL), jnp.float32),
            pltpu.SemaphoreType.DMA,
        ],
        compiler_params=pltpu.CompilerParams(use_tc_tiling_on_sc=False),
    )
    return f(*xi_qs, *xu_qs, gu, gv, ku, kv, zf)


def _tc_body(f_ref, cnt_ref, w_ref, out_ref):
    acc = jnp.zeros(out_ref.shape, jnp.float32)
    for r in range(R):
        c = cnt_ref[r, :, 0:1]
        rc = jnp.where(c > 0.5, 1.0 / c, 0.0)
        acc = acc + jnp.dot(f_ref[r] * rc, w_ref[r],
                            preferred_element_type=jnp.float32)
    out_ref[...] = acc


def _tc_project(f, cnt, weight, n):
    blk = 1000
    grid = (n // blk,)
    return pl.pallas_call(
        _tc_body,
        grid=grid,
        in_specs=[
            pl.BlockSpec((R, blk, IN_FEAT), lambda g: (0, g, 0)),
            pl.BlockSpec((R, blk, 16), lambda g: (0, g, 0)),
            pl.BlockSpec((R, IN_FEAT, HID), lambda g: (0, 0, 0)),
        ],
        out_specs=pl.BlockSpec((blk, HID), lambda g: (g, 0)),
        out_shape=jax.ShapeDtypeStruct((n, HID), jnp.float32),
    )(f, cnt, weight)


def kernel(x_user, x_item, u_s, v_s, rate, weight):
    E = u_s.shape[0]
    nchunk = -(-E // (NS * CHUNK))
    e_pad = NS * nchunk * CHUNK
    pad = e_pad - E

    u_s = u_s.astype(jnp.int32)
    v_s = v_s.astype(jnp.int32)
    rate = rate.astype(jnp.int32)

    key_u = rate * NU + u_s
    key_v = rate * NI + v_s
    trash = jnp.full((pad,), R * NU, jnp.int32)
    zero_idx = jnp.zeros((pad,), jnp.int32)
    ku = jnp.concatenate([key_u, trash]).reshape(NS, nchunk, CHUNK)
    kv = jnp.concatenate([key_v, trash]).reshape(NS, nchunk, CHUNK)
    gu = jnp.concatenate([v_s, zero_idx]).reshape(NS, nchunk, CHUNK)
    gv = jnp.concatenate([u_s, zero_idx]).reshape(NS, nchunk, CHUNK)

    xi_qs = [x_item[:, q * QCOL:(q + 1) * QCOL] for q in range(NQ)]
    xu_qs = [x_user[:, q * QCOL:(q + 1) * QCOL] for q in range(NQ)]
    ones_i = jnp.ones((CHUNK, 16), jnp.float32)
    zf = jnp.zeros((ROWS_PT, QCOL), jnp.float32)
    zc = jnp.zeros((ROWS_PT, 16), jnp.float32)

    ouc, ovc = _sc_counts(ku, kv, ones_i, zc)
    ou, ov = _sc_feats(xi_qs, xu_qs, gu, gv, ku, kv, zf)

    fu = jnp.concatenate([ou[q, :R * NU] for q in range(NQ)],
                         axis=-1).reshape(R, NU, IN_FEAT)
    fv = jnp.concatenate([ov[q, :R * NI] for q in range(NQ)],
                         axis=-1).reshape(R, NI, IN_FEAT)
    cu = ouc[:R * NU].reshape(R, NU, 16)
    cv = ovc[:R * NI].reshape(R, NI, 16)

    h_u = _tc_project(fu, cu, weight, NU)
    h_v = _tc_project(fv, cv, weight, NI)
    return h_u, h_v


# speedup vs baseline: 22.4817x; 1.4936x over previous
import functools

import jax
import jax.numpy as jnp
from jax import lax
from jax.experimental import pallas as pl
from jax.experimental.pallas import tpu as pltpu
from jax.experimental.pallas import tpu_sc as plsc

NU = 5000
NI = 5000
R = 5
IN_FEAT = 128
HID = 128
HALF = 64
QCOL = 32
NQ = IN_FEAT // QCOL

NC = 2
NS = 16
CHUNK = 128

ACC_ROWS = R * NU + 88
ROWS_PT = ACC_ROWS // NS


def _sc_mesh():
    return plsc.VectorSubcoreMesh(
        core_axis_name="c", subcore_axis_name="s", num_cores=NC, num_subcores=NS)


NBUF = 3


def _sc_accumulate(xi_qs, xu_qs, gu, gv, ku, kv, ones_i, zf, zc):
    nchunk = gu.shape[1]
    nq = len(xi_qs)

    def body(*refs):
        xi_t = refs[0:nq]
        xu_t = refs[nq:2 * nq]
        gu_h, gv_h, ku_h, kv_h, ones_h, zf_h, zc_h = refs[2 * nq:2 * nq + 7]
        ou, ouc, ov, ovc = refs[2 * nq + 7:2 * nq + 11]
        keys_t, gidx_t, rows_t, ones_t, acc_f, acc_c, sem_g, sem_s = refs[2 * nq + 11:]
        cid = lax.axis_index("c")
        sid = lax.axis_index("s")
        base = sid * ROWS_PT

        def run_side(tabs, gidx_hbm, keys_hbm, o, oc):
            pltpu.sync_copy(keys_hbm.at[sid], keys_t)
            pltpu.sync_copy(gidx_hbm.at[sid], gidx_t)
            pltpu.sync_copy(ones_h, ones_t)
            pltpu.sync_copy(zc_h, acc_c.at[pl.ds(base, ROWS_PT)])

            for q in range(nq):
                tab = tabs[q]
                pltpu.sync_copy(zf_h, acc_f.at[pl.ds(base, ROWS_PT)])
                plsc.subcore_barrier()

                for b in range(NBUF):
                    pltpu.async_copy(tab.at[gidx_t.at[b]], rows_t.at[b], sem_g)

                def chunk(j, carry):
                    slot = lax.rem(j, NBUF)
                    pltpu.make_async_copy(
                        tab.at[gidx_t.at[j]], rows_t.at[slot], sem_g).wait()
                    pltpu.async_copy(
                        rows_t.at[slot], acc_f.at[keys_t.at[j]], sem_s, add=True)
                    if q == 0:
                        pltpu.sync_copy(ones_t, acc_c.at[keys_t.at[j]], add=True)

                    @pl.when(jnp.logical_and(j >= 1, j - 1 + NBUF < nchunk))
                    def _():
                        pslot = lax.rem(j - 1, NBUF)
                        pltpu.make_async_copy(
                            rows_t.at[pslot], acc_f.at[keys_t.at[j - 1]], sem_s).wait()
                        pltpu.async_copy(
                            tab.at[gidx_t.at[j - 1 + NBUF]], rows_t.at[pslot], sem_g)

                    return carry

                lax.fori_loop(0, nchunk, chunk, 0)
                for b in range(NBUF):
                    pltpu.make_async_copy(
                        rows_t.at[b], acc_f.at[keys_t.at[0]], sem_s).wait()
                plsc.subcore_barrier()
                pltpu.sync_copy(acc_f.at[pl.ds(base, ROWS_PT)],
                                o.at[q, pl.ds(base, ROWS_PT)])
                if q == 0:
                    pltpu.sync_copy(acc_c.at[pl.ds(base, ROWS_PT)],
                                    oc.at[pl.ds(base, ROWS_PT)])
                plsc.subcore_barrier()

        @pl.when(cid == 0)
        def _():
            run_side(xi_t, gu_h, ku_h, ou, ouc)

        @pl.when(cid == 1)
        def _():
            run_side(xu_t, gv_h, kv_h, ov, ovc)

    f = pl.kernel(
        body,
        out_type=[
            jax.ShapeDtypeStruct((nq, ACC_ROWS, QCOL), jnp.float32),
            jax.ShapeDtypeStruct((ACC_ROWS, 16), jnp.float32),
            jax.ShapeDtypeStruct((nq, ACC_ROWS, QCOL), jnp.float32),
            jax.ShapeDtypeStruct((ACC_ROWS, 16), jnp.float32),
        ],
        mesh=_sc_mesh(),
        scratch_types=[
            pltpu.VMEM((nchunk, CHUNK), jnp.int32),
            pltpu.VMEM((nchunk, CHUNK), jnp.int32),
            pltpu.VMEM((NBUF, CHUNK, QCOL), jnp.float32),
            pltpu.VMEM((CHUNK, 16), jnp.float32),
            pltpu.VMEM_SHARED((ACC_ROWS, QCOL), jnp.float32),
            pltpu.VMEM_SHARED((ACC_ROWS, 16), jnp.float32),
            pltpu.SemaphoreType.DMA,
            pltpu.SemaphoreType.DMA,
        ],
        compiler_params=pltpu.CompilerParams(use_tc_tiling_on_sc=False),
    )
    return f(*xi_qs, *xu_qs, gu, gv, ku, kv, ones_i, zf, zc)


def _tc_body(f_ref, cnt_ref, w_ref, out_ref):
    acc = jnp.zeros(out_ref.shape, jnp.float32)
    for r in range(R):
        c = cnt_ref[r, :, 0:1]
        rc = jnp.where(c > 0.5, 1.0 / c, 0.0)
        acc = acc + jnp.dot(f_ref[r] * rc, w_ref[r],
                            preferred_element_type=jnp.float32)
    out_ref[...] = acc


def _tc_project(f, cnt, weight, n):
    blk = 1000
    grid = (n // blk,)
    return pl.pallas_call(
        _tc_body,
        grid=grid,
        in_specs=[
            pl.BlockSpec((R, blk, IN_FEAT), lambda g: (0, g, 0)),
            pl.BlockSpec((R, blk, 16), lambda g: (0, g, 0)),
            pl.BlockSpec((R, IN_FEAT, HID), lambda g: (0, 0, 0)),
        ],
        out_specs=pl.BlockSpec((blk, HID), lambda g: (g, 0)),
        out_shape=jax.ShapeDtypeStruct((n, HID), jnp.float32),
    )(f, cnt, weight)


def kernel(x_user, x_item, u_s, v_s, rate, weight):
    E = u_s.shape[0]
    nchunk = -(-E // (NS * CHUNK))
    e_pad = NS * nchunk * CHUNK
    pad = e_pad - E

    u_s = u_s.astype(jnp.int32)
    v_s = v_s.astype(jnp.int32)
    rate = rate.astype(jnp.int32)

    key_u = rate * NU + u_s
    key_v = rate * NI + v_s
    trash = jnp.full((pad,), R * NU, jnp.int32)
    zero_idx = jnp.zeros((pad,), jnp.int32)
    ku = jnp.concatenate([key_u, trash]).reshape(NS, nchunk, CHUNK)
    kv = jnp.concatenate([key_v, trash]).reshape(NS, nchunk, CHUNK)
    gu = jnp.concatenate([v_s, zero_idx]).reshape(NS, nchunk, CHUNK)
    gv = jnp.concatenate([u_s, zero_idx]).reshape(NS, nchunk, CHUNK)

    xi_qs = [x_item[:, q * QCOL:(q + 1) * QCOL] for q in range(NQ)]
    xu_qs = [x_user[:, q * QCOL:(q + 1) * QCOL] for q in range(NQ)]
    ones_i = jnp.ones((CHUNK, 16), jnp.float32)
    zf = jnp.zeros((ROWS_PT, QCOL), jnp.float32)
    zc = jnp.zeros((ROWS_PT, 16), jnp.float32)

    ou, ouc, ov, ovc = _sc_accumulate(
        xi_qs, xu_qs, gu, gv, ku, kv, ones_i, zf, zc)

    fu = jnp.concatenate([ou[q, :R * NU] for q in range(NQ)],
                         axis=-1).reshape(R, NU, IN_FEAT)
    fv = jnp.concatenate([ov[q, :R * NI] for q in range(NQ)],
                         axis=-1).reshape(R, NI, IN_FEAT)
    cu = ouc[:R * NU].reshape(R, NU, 16)
    cv = ovc[:R * NI].reshape(R, NI, 16)

    h_u = _tc_project(fu, cu, weight, NU)
    h_v = _tc_project(fv, cv, weight, NI)
    return h_u, h_v


# speedup vs baseline: 32.1515x; 1.4301x over previous
import functools

import jax
import jax.numpy as jnp
from jax import lax
from jax.experimental import pallas as pl
from jax.experimental.pallas import tpu as pltpu
from jax.experimental.pallas import tpu_sc as plsc

NU = 5000
NI = 5000
R = 5
IN_FEAT = 128
HID = 128
HALF = 64
QCOL = 32
NQ = IN_FEAT // QCOL

NC = 2
NS = 16
CHUNK = 128

ACC_ROWS = R * NU + 88
ROWS_PT = ACC_ROWS // NS


def _sc_mesh():
    return plsc.VectorSubcoreMesh(
        core_axis_name="c", subcore_axis_name="s", num_cores=NC, num_subcores=NS)


NBUF = 3


def _sc_accumulate(xi_qs, xu_qs, gu, gv, ku, kv, ones_i, zf, zc):
    nchunk = gu.shape[1]
    nq = len(xi_qs)

    def body(*refs):
        xi_t = refs[0:nq]
        xu_t = refs[nq:2 * nq]
        gu_h, gv_h, ku_h, kv_h, ones_h, zf_h, zc_h = refs[2 * nq:2 * nq + 7]
        ou, ouc, ov, ovc = refs[2 * nq + 7:2 * nq + 11]
        (keys_t, gidx_t, rows_t, ones_t, acc_f, acc_c,
         sem_g, sem_s, sem_c) = refs[2 * nq + 11:]
        cid = lax.axis_index("c")
        sid = lax.axis_index("s")
        base = sid * ROWS_PT
        tail_rows = R * NU - (NS - 1) * ROWS_PT

        def dump(src, dst_cols):
            @pl.when(sid < NS - 1)
            def _():
                pltpu.sync_copy(src.at[pl.ds(base, ROWS_PT)],
                                dst_cols(pl.ds(base, ROWS_PT)))

            @pl.when(sid == NS - 1)
            def _():
                pltpu.sync_copy(src.at[pl.ds(base, tail_rows)],
                                dst_cols(pl.ds(base, tail_rows)))

        def run_side(tabs, gidx_hbm, keys_hbm, o, oc):
            pltpu.sync_copy(keys_hbm.at[sid], keys_t)
            pltpu.sync_copy(gidx_hbm.at[sid], gidx_t)
            pltpu.sync_copy(ones_h, ones_t)
            pltpu.sync_copy(zc_h, acc_c.at[pl.ds(base, ROWS_PT)])

            for q in range(nq):
                tab = tabs[q]
                pltpu.sync_copy(zf_h, acc_f.at[pl.ds(base, ROWS_PT)])
                plsc.subcore_barrier()

                for b in range(NBUF):
                    pltpu.async_copy(tab.at[gidx_t.at[b]], rows_t.at[b], sem_g)

                def chunk(j, carry):
                    slot = lax.rem(j, NBUF)
                    pltpu.make_async_copy(
                        tab.at[gidx_t.at[j]], rows_t.at[slot], sem_g).wait()
                    pltpu.async_copy(
                        rows_t.at[slot], acc_f.at[keys_t.at[j]], sem_s, add=True)
                    if q == 0:
                        pltpu.async_copy(
                            ones_t, acc_c.at[keys_t.at[j]], sem_c, add=True)

                    @pl.when(jnp.logical_and(j >= 1, j - 1 + NBUF < nchunk))
                    def _():
                        pslot = lax.rem(j - 1, NBUF)
                        pltpu.make_async_copy(
                            rows_t.at[pslot], acc_f.at[keys_t.at[j - 1]], sem_s).wait()
                        pltpu.async_copy(
                            tab.at[gidx_t.at[j - 1 + NBUF]], rows_t.at[pslot], sem_g)

                    if q == 0:
                        @pl.when(j >= 1)
                        def _():
                            pltpu.make_async_copy(
                                ones_t, acc_c.at[keys_t.at[j - 1]], sem_c).wait()

                    return carry

                lax.fori_loop(0, nchunk, chunk, 0)
                for b in range(NBUF):
                    pltpu.make_async_copy(
                        rows_t.at[b], acc_f.at[keys_t.at[0]], sem_s).wait()
                if q == 0:
                    pltpu.make_async_copy(
                        ones_t, acc_c.at[keys_t.at[0]], sem_c).wait()
                plsc.subcore_barrier()
                dump(acc_f, lambda rs, q=q: o.at[rs, pl.ds(q * QCOL, QCOL)])
                if q == 0:
                    dump(acc_c, lambda rs: oc.at[rs])
                plsc.subcore_barrier()

        @pl.when(cid == 0)
        def _():
            run_side(xi_t, gu_h, ku_h, ou, ouc)

        @pl.when(cid == 1)
        def _():
            run_side(xu_t, gv_h, kv_h, ov, ovc)

    f = pl.kernel(
        body,
        out_type=[
            jax.ShapeDtypeStruct((R * NU, IN_FEAT), jnp.float32),
            jax.ShapeDtypeStruct((R * NU, 16), jnp.float32),
            jax.ShapeDtypeStruct((R * NI, IN_FEAT), jnp.float32),
            jax.ShapeDtypeStruct((R * NI, 16), jnp.float32),
        ],
        mesh=_sc_mesh(),
        scratch_types=[
            pltpu.VMEM((nchunk, CHUNK), jnp.int32),
            pltpu.VMEM((nchunk, CHUNK), jnp.int32),
            pltpu.VMEM((NBUF, CHUNK, QCOL), jnp.float32),
            pltpu.VMEM((CHUNK, 16), jnp.float32),
            pltpu.VMEM_SHARED((ACC_ROWS, QCOL), jnp.float32),
            pltpu.VMEM_SHARED((ACC_ROWS, 16), jnp.float32),
            pltpu.SemaphoreType.DMA,
            pltpu.SemaphoreType.DMA,
            pltpu.SemaphoreType.DMA,
        ],
        compiler_params=pltpu.CompilerParams(use_tc_tiling_on_sc=False),
    )
    return f(*xi_qs, *xu_qs, gu, gv, ku, kv, ones_i, zf, zc)


def _tc_body(f_ref, cnt_ref, w_ref, out_ref):
    acc = jnp.zeros(out_ref.shape, jnp.float32)
    for r in range(R):
        c = cnt_ref[r, :, 0:1]
        rc = jnp.where(c > 0.5, 1.0 / c, 0.0)
        acc = acc + jnp.dot(f_ref[r] * rc, w_ref[r],
                            preferred_element_type=jnp.float32)
    out_ref[...] = acc


def _tc_project(f, cnt, weight, n):
    blk = 1000
    grid = (n // blk,)
    return pl.pallas_call(
        _tc_body,
        grid=grid,
        in_specs=[
            pl.BlockSpec((R, blk, IN_FEAT), lambda g: (0, g, 0)),
            pl.BlockSpec((R, blk, 16), lambda g: (0, g, 0)),
            pl.BlockSpec((R, IN_FEAT, HID), lambda g: (0, 0, 0)),
        ],
        out_specs=pl.BlockSpec((blk, HID), lambda g: (g, 0)),
        out_shape=jax.ShapeDtypeStruct((n, HID), jnp.float32),
    )(f, cnt, weight)


def kernel(x_user, x_item, u_s, v_s, rate, weight):
    E = u_s.shape[0]
    nchunk = -(-E // (NS * CHUNK))
    e_pad = NS * nchunk * CHUNK
    pad = e_pad - E

    u_s = u_s.astype(jnp.int32)
    v_s = v_s.astype(jnp.int32)
    rate = rate.astype(jnp.int32)

    key_u = rate * NU + u_s
    key_v = rate * NI + v_s
    trash = jnp.full((pad,), R * NU, jnp.int32)
    zero_idx = jnp.zeros((pad,), jnp.int32)
    ku = jnp.concatenate([key_u, trash]).reshape(NS, nchunk, CHUNK)
    kv = jnp.concatenate([key_v, trash]).reshape(NS, nchunk, CHUNK)
    gu = jnp.concatenate([v_s, zero_idx]).reshape(NS, nchunk, CHUNK)
    gv = jnp.concatenate([u_s, zero_idx]).reshape(NS, nchunk, CHUNK)

    xi_qs = [x_item[:, q * QCOL:(q + 1) * QCOL] for q in range(NQ)]
    xu_qs = [x_user[:, q * QCOL:(q + 1) * QCOL] for q in range(NQ)]
    ones_i = jnp.ones((CHUNK, 16), jnp.float32)
    zf = jnp.zeros((ROWS_PT, QCOL), jnp.float32)
    zc = jnp.zeros((ROWS_PT, 16), jnp.float32)

    ou, ouc, ov, ovc = _sc_accumulate(
        xi_qs, xu_qs, gu, gv, ku, kv, ones_i, zf, zc)

    fu = ou.reshape(R, NU, IN_FEAT)
    fv = ov.reshape(R, NI, IN_FEAT)
    cu = ouc.reshape(R, NU, 16)
    cv = ovc.reshape(R, NI, 16)

    h_u = _tc_project(fu, cu, weight, NU)
    h_v = _tc_project(fv, cv, weight, NI)
    return h_u, h_v


# speedup vs baseline: 35.3751x; 1.1003x over previous
import functools

import jax
import jax.numpy as jnp
from jax import lax
from jax.experimental import pallas as pl
from jax.experimental.pallas import tpu as pltpu
from jax.experimental.pallas import tpu_sc as plsc

NU = 5000
NI = 5000
R = 5
IN_FEAT = 128
HID = 128
HALF = 64
QCOL = 32
NQ = IN_FEAT // QCOL

NC = 2
NS = 16
CHUNK = 128

ACC_ROWS = R * NU + 88
ROWS_PT = ACC_ROWS // NS


def _sc_mesh():
    return plsc.VectorSubcoreMesh(
        core_axis_name="c", subcore_axis_name="s", num_cores=NC, num_subcores=NS)


NBUF = 8
LAG = 4


def _sc_accumulate(xi_qs, xu_qs, gu, gv, ku, kv, ones_i, zf):
    nchunk = gu.shape[1]
    nq = len(xi_qs)

    def body(*refs):
        xi_t = refs[0:nq]
        xu_t = refs[nq:2 * nq]
        gu_h, gv_h, ku_h, kv_h, ones_h, zf_h = refs[2 * nq:2 * nq + 6]
        ou, ouc, ov, ovc = refs[2 * nq + 6:2 * nq + 10]
        (keys_t, gidx_t, rows_t, ones_t, acc_f,
         sem_g, sem_s) = refs[2 * nq + 10:]
        cid = lax.axis_index("c")
        sid = lax.axis_index("s")
        base = sid * ROWS_PT
        tail_rows = R * NU - (NS - 1) * ROWS_PT

        def dump(src, dst_cols):
            @pl.when(sid < NS - 1)
            def _():
                pltpu.sync_copy(src.at[pl.ds(base, ROWS_PT)],
                                dst_cols(pl.ds(base, ROWS_PT)))

            @pl.when(sid == NS - 1)
            def _():
                pltpu.sync_copy(src.at[pl.ds(base, tail_rows)],
                                dst_cols(pl.ds(base, tail_rows)))

        def run_side(tabs, gidx_hbm, keys_hbm, o, oc):
            pltpu.sync_copy(keys_hbm.at[sid], keys_t)
            pltpu.sync_copy(gidx_hbm.at[sid], gidx_t)
            pltpu.sync_copy(ones_h, ones_t)

            pltpu.sync_copy(zf_h, acc_f.at[pl.ds(base, ROWS_PT)])
            plsc.subcore_barrier()

            def cchunk(j, carry):
                pltpu.async_copy(ones_t, acc_f.at[keys_t.at[j]], sem_s, add=True)

                @pl.when(j >= LAG)
                def _():
                    pltpu.make_async_copy(
                        ones_t, acc_f.at[keys_t.at[0]], sem_s).wait()

                return carry

            lax.fori_loop(0, nchunk, cchunk, 0)
            for _ in range(LAG):
                pltpu.make_async_copy(
                    ones_t, acc_f.at[keys_t.at[0]], sem_s).wait()
            plsc.subcore_barrier()
            dump(acc_f, lambda rs: oc.at[rs])
            plsc.subcore_barrier()

            for q in range(nq):
                tab = tabs[q]
                pltpu.sync_copy(zf_h, acc_f.at[pl.ds(base, ROWS_PT)])
                plsc.subcore_barrier()

                for b in range(LAG):
                    pltpu.async_copy(tab.at[gidx_t.at[b]], rows_t.at[b], sem_g)

                def chunk(j, carry):
                    slot = lax.rem(j, NBUF)
                    pltpu.make_async_copy(
                        tab.at[gidx_t.at[j]], rows_t.at[slot], sem_g).wait()
                    pltpu.async_copy(
                        rows_t.at[slot], acc_f.at[keys_t.at[j]], sem_s, add=True)

                    @pl.when(j + LAG < nchunk)
                    def _():
                        @pl.when(j >= LAG)
                        def _():
                            pltpu.make_async_copy(
                                rows_t.at[0], acc_f.at[keys_t.at[0]],
                                sem_s).wait()

                        pltpu.async_copy(
                            tab.at[gidx_t.at[j + LAG]],
                            rows_t.at[lax.rem(j + LAG, NBUF)], sem_g)

                    return carry

                lax.fori_loop(0, nchunk, chunk, 0)
                for b in range(NBUF):
                    pltpu.make_async_copy(
                        rows_t.at[b], acc_f.at[keys_t.at[0]], sem_s).wait()
                plsc.subcore_barrier()
                dump(acc_f, lambda rs, q=q: o.at[rs, pl.ds(q * QCOL, QCOL)])
                plsc.subcore_barrier()

        @pl.when(cid == 0)
        def _():
            run_side(xi_t, gu_h, ku_h, ou, ouc)

        @pl.when(cid == 1)
        def _():
            run_side(xu_t, gv_h, kv_h, ov, ovc)

    f = pl.kernel(
        body,
        out_type=[
            jax.ShapeDtypeStruct((R * NU, IN_FEAT), jnp.float32),
            jax.ShapeDtypeStruct((R * NU, QCOL), jnp.float32),
            jax.ShapeDtypeStruct((R * NI, IN_FEAT), jnp.float32),
            jax.ShapeDtypeStruct((R * NI, QCOL), jnp.float32),
        ],
        mesh=_sc_mesh(),
        scratch_types=[
            pltpu.VMEM((nchunk, CHUNK), jnp.int32),
            pltpu.VMEM((nchunk, CHUNK), jnp.int32),
            pltpu.VMEM((NBUF, CHUNK, QCOL), jnp.float32),
            pltpu.VMEM((CHUNK, QCOL), jnp.float32),
            pltpu.VMEM_SHARED((ACC_ROWS, QCOL), jnp.float32),
            pltpu.SemaphoreType.DMA,
            pltpu.SemaphoreType.DMA,
        ],
        compiler_params=pltpu.CompilerParams(use_tc_tiling_on_sc=False),
    )
    return f(*xi_qs, *xu_qs, gu, gv, ku, kv, ones_i, zf)


def _tc_body(f_ref, cnt_ref, w_ref, out_ref):
    acc = jnp.zeros(out_ref.shape, jnp.float32)
    for r in range(R):
        c = cnt_ref[r, :, 0:1]
        rc = jnp.where(c > 0.5, 1.0 / c, 0.0)
        acc = acc + jnp.dot(f_ref[r] * rc, w_ref[r],
                            preferred_element_type=jnp.float32)
    out_ref[...] = acc


def _tc_project(f, cnt, weight, n):
    blk = 1000
    grid = (n // blk,)
    return pl.pallas_call(
        _tc_body,
        grid=grid,
        in_specs=[
            pl.BlockSpec((R, blk, IN_FEAT), lambda g: (0, g, 0)),
            pl.BlockSpec((R, blk, QCOL), lambda g: (0, g, 0)),
            pl.BlockSpec((R, IN_FEAT, HID), lambda g: (0, 0, 0)),
        ],
        out_specs=pl.BlockSpec((blk, HID), lambda g: (g, 0)),
        out_shape=jax.ShapeDtypeStruct((n, HID), jnp.float32),
    )(f, cnt, weight)


def kernel(x_user, x_item, u_s, v_s, rate, weight):
    E = u_s.shape[0]
    nchunk = -(-E // (NS * CHUNK))
    e_pad = NS * nchunk * CHUNK
    pad = e_pad - E

    u_s = u_s.astype(jnp.int32)
    v_s = v_s.astype(jnp.int32)
    rate = rate.astype(jnp.int32)

    key_u = rate * NU + u_s
    key_v = rate * NI + v_s
    trash = jnp.full((pad,), R * NU, jnp.int32)
    zero_idx = jnp.zeros((pad,), jnp.int32)
    ku = jnp.concatenate([key_u, trash]).reshape(NS, nchunk, CHUNK)
    kv = jnp.concatenate([key_v, trash]).reshape(NS, nchunk, CHUNK)
    gu = jnp.concatenate([v_s, zero_idx]).reshape(NS, nchunk, CHUNK)
    gv = jnp.concatenate([u_s, zero_idx]).reshape(NS, nchunk, CHUNK)

    xi_qs = [x_item[:, q * QCOL:(q + 1) * QCOL] for q in range(NQ)]
    xu_qs = [x_user[:, q * QCOL:(q + 1) * QCOL] for q in range(NQ)]
    ones_i = jnp.ones((CHUNK, QCOL), jnp.float32)
    zf = jnp.zeros((ROWS_PT, QCOL), jnp.float32)

    ou, ouc, ov, ovc = _sc_accumulate(
        xi_qs, xu_qs, gu, gv, ku, kv, ones_i, zf)

    fu = ou.reshape(R, NU, IN_FEAT)
    fv = ov.reshape(R, NI, IN_FEAT)
    cu = ouc.reshape(R, NU, QCOL)
    cv = ovc.reshape(R, NI, QCOL)

    h_u = _tc_project(fu, cu, weight, NU)
    h_v = _tc_project(fv, cv, weight, NI)
    return h_u, h_v
